# baseline (speedup 1.0000x reference)
import jax
import jax.numpy as jnp
from jax import lax
from jax.experimental import pallas as pl
from jax.experimental.pallas import tpu as pltpu
from jax.experimental.pallas import tpu_sc as plsc

HIDDEN = 1024
EXPERTS = 64
TOPK = 8
TOKENS = 32768
BLOCK = 512
HALF = BLOCK // 2

SLICES = 2
TOK_S = TOKENS // SLICES
BLOCKS_S = TOK_S // BLOCK

NUM_CORES = 2
NUM_SUBCORES = 16
LANES = 16
NW = NUM_CORES * NUM_SUBCORES
ROWS_PER_W = TOK_S // NW
CHUNK = 512
GROUPS = CHUNK // LANES
NCHUNK = ROWS_PER_W // CHUNK


def _logits_kernel(x_ref, w_ref, b_ref, gin_ref, logits_ref, packed_ref):
    del gin_ref
    x = x_ref[...]
    w = w_ref[...]
    acc = jax.lax.dot_general(
        x, w, (((1,), (1,)), ((), ())), preferred_element_type=jnp.float32
    )
    logits = acc + b_ref[...]
    logits_ref[...] = logits
    packed_ref[...] = jnp.concatenate([logits[:HALF], logits[HALF:]], axis=1)


def _compute_logits_slice(x, W, b2, gate_in, s):
    off = s * BLOCKS_S
    return pl.pallas_call(
        _logits_kernel,
        grid=(BLOCKS_S,),
        in_specs=[
            pl.BlockSpec((BLOCK, HIDDEN), lambda i: (i + off, 0)),
            pl.BlockSpec((EXPERTS, HIDDEN), lambda i: (0, 0)),
            pl.BlockSpec((1, EXPERTS), lambda i: (0, 0)),
            pl.BlockSpec((BLOCK, EXPERTS), lambda i: (i + off, 0)),
        ],
        out_specs=[
            pl.BlockSpec((BLOCK, EXPERTS), lambda i: (i + off, 0)),
            pl.BlockSpec((HALF, 2 * EXPERTS), lambda i: (i, 0)),
        ],
        out_shape=[
            jax.ShapeDtypeStruct((TOKENS, EXPERTS), jnp.float32),
            jax.ShapeDtypeStruct((TOK_S // 2, 2 * EXPERTS), jnp.float32),
        ],
        input_output_aliases={3: 0},
        compiler_params=pltpu.CompilerParams(
            dimension_semantics=("arbitrary",),
        ),
    )(x, W, b2, gate_in)


def _route_body(logits_hbm, sparse_hbm, idx_hbm, in_v, out_v, idx_v):
    c = lax.axis_index("c")
    s = lax.axis_index("s")
    wid = s * NUM_CORES + c
    base = wid * ROWS_PER_W
    lane_iota = lax.iota(jnp.int32, LANES)
    zero16 = jnp.zeros((LANES,), jnp.float32)
    neg16 = jnp.full((LANES,), -jnp.inf, jnp.float32)
    izero16 = jnp.zeros((LANES,), jnp.int32)

    def chunk_body(ci, carry):
        row0 = base + ci * CHUNK
        pltpu.sync_copy(
            logits_hbm.at[pl.ds(row0 * EXPERTS, CHUNK * EXPERTS)], in_v
        )

        def zero_body(z, _):
            for u in range(8):
                out_v[pl.ds(z * 128 + u * LANES, LANES)] = zero16
            return 0

        lax.fori_loop(0, CHUNK * EXPERTS // 128, zero_body, 0)

        def group_body(g, carry2):
            rbase = ((g & 15) * LANES + lane_iota) * (2 * EXPERTS) + (
                g >> 4
            ) * EXPERTS

            UNROLL = 8

            def exp_body(eo, tk):
                vs = list(tk[:TOPK])
                ix = list(tk[TOPK:])
                ebase = izero16 + eo * UNROLL
                for k in range(UNROLL):
                    t = plsc.load_gather(in_v, [rbase + (eo * UNROLL + k)])
                    ti = ebase + k
                    for j in range(TOPK):
                        cgt = t > vs[j]
                        nv = jnp.maximum(vs[j], t)
                        nt = jnp.minimum(vs[j], t)
                        ni = jnp.where(cgt, ti, ix[j])
                        nti = jnp.where(cgt, ix[j], ti)
                        vs[j], t, ix[j], ti = nv, nt, ni, nti
                return tuple(vs) + tuple(ix)

            init = tuple([neg16] * TOPK) + tuple([izero16] * TOPK)
            tk = lax.fori_loop(0, EXPERTS // UNROLL, exp_body, init)
            vs = tk[:TOPK]
            ix = tk[TOPK:]

            m0 = vs[0]
            es = [jnp.exp(v - m0) for v in vs]
            tot = es[0]
            for j in range(1, TOPK):
                tot = tot + es[j]
            inv = 1.0 / tot
            kbase = ((g & 15) * LANES + lane_iota) * (2 * TOPK) + (
                g >> 4
            ) * TOPK
            for j in range(TOPK):
                plsc.store_scatter(out_v, [rbase + ix[j]], es[j] * inv)
                plsc.store_scatter(idx_v, [kbase + j], ix[j])
            return carry2

        lax.fori_loop(0, GROUPS, group_body, 0)
        pltpu.sync_copy(
            out_v, sparse_hbm.at[pl.ds(row0 * EXPERTS, CHUNK * EXPERTS)]
        )
        pltpu.sync_copy(idx_v, idx_hbm.at[pl.ds(row0 * TOPK, CHUNK * TOPK)])
        return carry

    lax.fori_loop(0, NCHUNK, chunk_body, 0)


def _route(logits_flat):
    mesh = plsc.VectorSubcoreMesh(
        core_axis_name="c",
        subcore_axis_name="s",
        num_cores=NUM_CORES,
        num_subcores=NUM_SUBCORES,
    )
    fn = pl.kernel(
        _route_body,
        out_type=[
            jax.ShapeDtypeStruct((TOK_S * EXPERTS,), jnp.float32),
            jax.ShapeDtypeStruct((TOK_S * TOPK,), jnp.int32),
        ],
        mesh=mesh,
        scratch_types=[
            pltpu.VMEM((CHUNK * EXPERTS,), jnp.float32),
            pltpu.VMEM((CHUNK * EXPERTS,), jnp.float32),
            pltpu.VMEM((CHUNK * TOPK,), jnp.int32),
        ],
        compiler_params=pltpu.CompilerParams(needs_layout_passes=False),
    )
    return fn(logits_flat)


def _unpack_kernel(cp_ref, icp_ref, sin_ref, iin_ref, sparse_ref, idx_ref):
    del sin_ref, iin_ref
    cp = cp_ref[...]
    icp = icp_ref[...]
    sparse_ref[...] = jnp.concatenate(
        [cp[:, :EXPERTS], cp[:, EXPERTS:]], axis=0
    )
    idx_ref[...] = jnp.concatenate([icp[:, :TOPK], icp[:, TOPK:]], axis=0)


def _unpack_slice(sparse_cp, idx_cp, sparse_in, idx_in, s):
    off = s * BLOCKS_S
    return pl.pallas_call(
        _unpack_kernel,
        grid=(BLOCKS_S,),
        in_specs=[
            pl.BlockSpec((HALF, 2 * EXPERTS), lambda i: (i, 0)),
            pl.BlockSpec((HALF, 2 * TOPK), lambda i: (i, 0)),
            pl.BlockSpec((BLOCK, EXPERTS), lambda i: (i + off, 0)),
            pl.BlockSpec((BLOCK, TOPK), lambda i: (i + off, 0)),
        ],
        out_specs=[
            pl.BlockSpec((BLOCK, EXPERTS), lambda i: (i + off, 0)),
            pl.BlockSpec((BLOCK, TOPK), lambda i: (i + off, 0)),
        ],
        out_shape=[
            jax.ShapeDtypeStruct((TOKENS, EXPERTS), jnp.float32),
            jax.ShapeDtypeStruct((TOKENS, TOPK), jnp.int32),
        ],
        input_output_aliases={2: 0, 3: 1},
        compiler_params=pltpu.CompilerParams(
            dimension_semantics=("arbitrary",),
        ),
    )(sparse_cp, idx_cp, sparse_in, idx_in)


@jax.jit
def kernel(x, W, b):
    b2 = b.reshape(1, EXPERTS)
    gate = jnp.zeros((TOKENS, EXPERTS), jnp.float32)
    packed = []
    for s in range(SLICES):
        gate, p_s = _compute_logits_slice(x, W, b2, gate, s)
        packed.append(p_s)
    routed = [_route(p.reshape(-1)) for p in packed]
    sparse = jnp.zeros((TOKENS, EXPERTS), jnp.float32)
    idx = jnp.zeros((TOKENS, TOPK), jnp.int32)
    for s in range(SLICES):
        sparse, idx = _unpack_slice(
            routed[s][0].reshape(TOK_S // 2, 2 * EXPERTS),
            routed[s][1].reshape(TOK_S // 2, 2 * TOPK),
            sparse,
            idx,
            s,
        )
    return sparse, idx, gate


# speedup vs baseline: 1.3074x; 1.3074x over previous
import jax
import jax.numpy as jnp
from jax import lax
from jax.experimental import pallas as pl
from jax.experimental.pallas import tpu as pltpu
from jax.experimental.pallas import tpu_sc as plsc

HIDDEN = 1024
EXPERTS = 64
TOPK = 8
TOKENS = 32768
BLOCK = 512
HALF_T = TOKENS // 2
NBLK = HALF_T // BLOCK

NUM_CORES = 2
NUM_SUBCORES = 16
LANES = 16
NW = NUM_CORES * NUM_SUBCORES
TOK_PER_W = TOKENS // NW
CHUNK = 512
GROUPS = CHUNK // LANES
NCHUNK = TOK_PER_W // CHUNK


def _logits_kernel(xl_ref, xr_ref, w_ref, b_ref, g_ref, packed_ref):
    w = w_ref[...]
    b = b_ref[...]
    accl = jax.lax.dot_general(
        xl_ref[...], w, (((1,), (1,)), ((), ())),
        preferred_element_type=jnp.float32,
    )
    accr = jax.lax.dot_general(
        xr_ref[...], w, (((1,), (1,)), ((), ())),
        preferred_element_type=jnp.float32,
    )
    ll = accl + b
    lr = accr + b
    g_ref[0] = ll
    g_ref[1] = lr
    packed_ref[...] = jnp.concatenate([ll, lr], axis=1)


def _compute_logits(x, W, b2):
    return pl.pallas_call(
        _logits_kernel,
        grid=(NBLK,),
        in_specs=[
            pl.BlockSpec((BLOCK, HIDDEN), lambda i: (i, 0)),
            pl.BlockSpec((BLOCK, HIDDEN), lambda i: (i + NBLK, 0)),
            pl.BlockSpec((EXPERTS, HIDDEN), lambda i: (0, 0)),
            pl.BlockSpec((1, EXPERTS), lambda i: (0, 0)),
        ],
        out_specs=[
            pl.BlockSpec((2, BLOCK, EXPERTS), lambda i: (0, i, 0)),
            pl.BlockSpec((BLOCK, 2 * EXPERTS), lambda i: (i, 0)),
        ],
        out_shape=[
            jax.ShapeDtypeStruct((2, HALF_T, EXPERTS), jnp.float32),
            jax.ShapeDtypeStruct((HALF_T, 2 * EXPERTS), jnp.float32),
        ],
        compiler_params=pltpu.CompilerParams(
            dimension_semantics=("arbitrary",),
        ),
    )(x, x, W, b2)


def _route_body(logits_hbm, sparse_hbm, idx_hbm, in_v, out_v, idx_v):
    c = lax.axis_index("c")
    s = lax.axis_index("s")
    wid = s * NUM_CORES + c
    base = wid * TOK_PER_W
    lane_iota = lax.iota(jnp.int32, LANES)
    zero16 = jnp.zeros((LANES,), jnp.float32)
    neg16 = jnp.full((LANES,), -jnp.inf, jnp.float32)
    izero16 = jnp.zeros((LANES,), jnp.int32)

    def chunk_body(ci, carry):
        row0 = base + ci * CHUNK
        pltpu.sync_copy(
            logits_hbm.at[pl.ds(row0 * EXPERTS, CHUNK * EXPERTS)], in_v
        )

        @plsc.parallel_loop(0, CHUNK * EXPERTS // 128, 1, unroll=4)
        def zero_body(z):
            for u in range(8):
                out_v[pl.ds(z * 128 + u * LANES, LANES)] = zero16

        @plsc.parallel_loop(0, GROUPS, 1, unroll=2)
        def group_body(g):
            rbase = ((g & 15) * LANES + lane_iota) * (2 * EXPERTS) + (
                g >> 4
            ) * EXPERTS

            UNROLL = 8

            def exp_body(eo, tk):
                vs = list(tk[:TOPK])
                ix = list(tk[TOPK:])
                ebase = izero16 + eo * UNROLL
                for k in range(UNROLL):
                    t = plsc.load_gather(in_v, [rbase + (eo * UNROLL + k)])
                    ti = ebase + k
                    for j in range(TOPK):
                        cgt = t > vs[j]
                        nv = jnp.maximum(vs[j], t)
                        nt = jnp.minimum(vs[j], t)
                        ni = jnp.where(cgt, ti, ix[j])
                        nti = jnp.where(cgt, ix[j], ti)
                        vs[j], t, ix[j], ti = nv, nt, ni, nti
                return tuple(vs) + tuple(ix)

            init = tuple([neg16] * TOPK) + tuple([izero16] * TOPK)
            tk = plsc.parallel_loop(0, EXPERTS // UNROLL, 1, carry=init)(
                exp_body
            )
            vs = tk[:TOPK]
            ix = tk[TOPK:]

            m0 = vs[0]
            es = [jnp.exp(v - m0) for v in vs]
            tot = es[0]
            for j in range(1, TOPK):
                tot = tot + es[j]
            inv = 1.0 / tot
            kbase = ((g & 15) * LANES + lane_iota) * (2 * TOPK) + (
                g >> 4
            ) * TOPK
            for j in range(TOPK):
                plsc.store_scatter(out_v, [rbase + ix[j]], es[j] * inv)
                plsc.store_scatter(idx_v, [kbase + j], ix[j])
        pltpu.sync_copy(
            out_v, sparse_hbm.at[pl.ds(row0 * EXPERTS, CHUNK * EXPERTS)]
        )
        pltpu.sync_copy(idx_v, idx_hbm.at[pl.ds(row0 * TOPK, CHUNK * TOPK)])
        return carry

    lax.fori_loop(0, NCHUNK, chunk_body, 0)


def _route(logits_flat):
    mesh = plsc.VectorSubcoreMesh(
        core_axis_name="c",
        subcore_axis_name="s",
        num_cores=NUM_CORES,
        num_subcores=NUM_SUBCORES,
    )
    fn = pl.kernel(
        _route_body,
        out_type=[
            jax.ShapeDtypeStruct((TOKENS * EXPERTS,), jnp.float32),
            jax.ShapeDtypeStruct((TOKENS * TOPK,), jnp.int32),
        ],
        mesh=mesh,
        scratch_types=[
            pltpu.VMEM((CHUNK * EXPERTS,), jnp.float32),
            pltpu.VMEM((CHUNK * EXPERTS,), jnp.float32),
            pltpu.VMEM((CHUNK * TOPK,), jnp.int32),
        ],
        compiler_params=pltpu.CompilerParams(needs_layout_passes=False),
    )
    return fn(logits_flat)


@jax.jit
def kernel(x, W, b):
    b2 = b.reshape(1, EXPERTS)
    gate3, packed = _compute_logits(x, W, b2)
    gate = gate3.reshape(TOKENS, EXPERTS)
    sp_flat, idx_flat = _route(packed.reshape(-1))
    sp2 = sp_flat.reshape(HALF_T, 2 * EXPERTS)
    ix2 = idx_flat.reshape(HALF_T, 2 * TOPK)
    sparse = jnp.concatenate([sp2[:, :EXPERTS], sp2[:, EXPERTS:]], axis=0)
    idx = jnp.concatenate([ix2[:, :TOPK], ix2[:, TOPK:]], axis=0)
    return sparse, idx, gate
